# Initial kernel scaffold; baseline (speedup 1.0000x reference)
import functools

import jax
import jax.numpy as jnp
from jax import lax
from jax.experimental import pallas as pl
from jax.experimental.pallas import tpu as pltpu
from jax.experimental.pallas import tpu_sc as plsc

N_CORES = 2
N_SUBCORES = 16
NW = N_CORES * N_SUBCORES
LANES = 16
CHUNK = 128


def _bcast(v16, lane):
    idx = jnp.full((LANES,), lane, dtype=jnp.int32)
    return jnp.take(v16, idx, mode="promise_in_bounds")


def _make_sc_aggregate(n_nodes, d, n_chunks):
    rows_per_tile = n_nodes // N_SUBCORES
    mesh = plsc.VectorSubcoreMesh(
        core_axis_name="c", subcore_axis_name="s",
        num_cores=N_CORES, num_subcores=N_SUBCORES)

    @functools.partial(
        pl.kernel,
        out_type=jax.ShapeDtypeStruct((N_CORES, n_nodes, d), jnp.float32),
        mesh=mesh,
        scratch_types=[
            pltpu.VMEM((n_chunks, CHUNK), jnp.int32),
            pltpu.VMEM((n_chunks, CHUNK), jnp.int32),
            pltpu.VMEM((n_chunks, CHUNK), jnp.float32),
            pltpu.VMEM((CHUNK, d), jnp.float32),
            pltpu.VMEM_SHARED((n_nodes, d), jnp.float32),
        ],
    )
    def sc_aggregate(x_hbm, col_hbm, row_hbm, val_hbm, zero_hbm, out_hbm,
                     col_v, row_v, val_v, rows_v, acc):
        cid = lax.axis_index("c")
        sid = lax.axis_index("s")
        wid = cid * N_SUBCORES + sid

        r0 = sid * rows_per_tile
        pltpu.sync_copy(zero_hbm.at[pl.ds(r0, rows_per_tile)],
                        acc.at[pl.ds(r0, rows_per_tile)])

        base = wid * n_chunks
        pltpu.sync_copy(col_hbm.at[pl.ds(base, n_chunks)], col_v)
        pltpu.sync_copy(row_hbm.at[pl.ds(base, n_chunks)], row_v)
        pltpu.sync_copy(val_hbm.at[pl.ds(base, n_chunks)], val_v)
        plsc.subcore_barrier()

        def chunk_body(j, carry):
            pltpu.sync_copy(x_hbm.at[col_v.at[j]], rows_v)
            vrow = val_v.at[j]
            for g in range(CHUNK // LANES):
                v16 = vrow[pl.ds(g * LANES, LANES)]
                for l in range(LANES):
                    e = g * LANES + l
                    bc = _bcast(v16, l)
                    for k in range(d // LANES):
                        sl = pl.ds(k * LANES, LANES)
                        rows_v[e, sl] = rows_v[e, sl] * bc
            pltpu.sync_copy(rows_v, acc.at[row_v.at[j]], add=True)
            return carry

        lax.fori_loop(0, n_chunks, chunk_body, 0)
        plsc.subcore_barrier()

        pltpu.sync_copy(acc.at[pl.ds(r0, rows_per_tile)],
                        out_hbm.at[cid, pl.ds(r0, rows_per_tile)])

    return sc_aggregate


def _tc_combine_matmul(partials, w):
    n_nodes, d = partials.shape[1], partials.shape[2]

    def body(p_ref, w_ref, o_ref):
        agg = p_ref[0] + p_ref[1]
        o_ref[...] = lax.dot_general(
            agg, w_ref[...], (((1,), (1,)), ((), ())),
            preferred_element_type=jnp.float32)

    return pl.pallas_call(
        body,
        out_shape=jax.ShapeDtypeStruct((n_nodes, d), jnp.float32),
    )(partials, w)


def kernel(_x, adj_indices, adj_values, W):
    n_nodes, d = _x.shape
    e = adj_values.shape[0]

    row = adj_indices[0].astype(jnp.int32)
    col = adj_indices[1].astype(jnp.int32)
    val = adj_values.astype(jnp.float32)

    per_tile = -(-e // (NW * CHUNK)) * CHUNK
    e_pad = per_tile * NW
    n_chunks = per_tile // CHUNK
    pad = e_pad - e
    if pad:
        row = jnp.concatenate([row, jnp.zeros((pad,), jnp.int32)])
        col = jnp.concatenate([col, jnp.zeros((pad,), jnp.int32)])
        val = jnp.concatenate([val, jnp.zeros((pad,), jnp.float32)])
    row2 = row.reshape(NW * n_chunks, CHUNK)
    col2 = col.reshape(NW * n_chunks, CHUNK)
    val2 = val.reshape(NW * n_chunks, CHUNK)
    zero = jnp.zeros((n_nodes, d), jnp.float32)

    sc_aggregate = _make_sc_aggregate(n_nodes, d, n_chunks)
    partials = sc_aggregate(_x, col2, row2, val2, zero)
    return _tc_combine_matmul(partials, W)


# speedup vs baseline: 2.9382x; 2.9382x over previous
import functools

import jax
import jax.numpy as jnp
from jax import lax
from jax.experimental import pallas as pl
from jax.experimental.pallas import tpu as pltpu
from jax.experimental.pallas import tpu_sc as plsc

N_CORES = 2
N_SUBCORES = 16
NW = N_CORES * N_SUBCORES
LANES = 16
CHUNK = 128


_GDN = lax.GatherDimensionNumbers(
    offset_dims=(), collapsed_slice_dims=(0,), start_index_map=(0,))


def _bcast(v16, lane):
    idx = jnp.full((LANES, 1), lane, dtype=jnp.int32)
    return lax.gather(v16, idx, _GDN, (1,),
                      mode=lax.GatherScatterMode.PROMISE_IN_BOUNDS)


def _make_sc_aggregate(n_pad, d, n_chunks):
    rows_per_tile = n_pad // N_SUBCORES
    mesh = plsc.VectorSubcoreMesh(
        core_axis_name="c", subcore_axis_name="s",
        num_cores=N_CORES, num_subcores=N_SUBCORES)

    @functools.partial(
        pl.kernel,
        out_type=jax.ShapeDtypeStruct((N_CORES, n_pad, d), jnp.float32),
        mesh=mesh,
        scratch_types=[
            pltpu.VMEM((n_chunks, CHUNK), jnp.int32),
            pltpu.VMEM((n_chunks, CHUNK), jnp.int32),
            pltpu.VMEM((n_chunks, CHUNK), jnp.float32),
            pltpu.VMEM((CHUNK, d), jnp.float32),
            pltpu.VMEM_SHARED((n_pad, d), jnp.float32),
        ],
    )
    def sc_aggregate(x_hbm, col_hbm, row_hbm, val_hbm, zero_hbm, out_hbm,
                     col_v, row_v, val_v, rows_v, acc):
        cid = lax.axis_index("c")
        sid = lax.axis_index("s")
        wid = cid * N_SUBCORES + sid

        r0 = sid * rows_per_tile
        pltpu.sync_copy(zero_hbm.at[pl.ds(r0, rows_per_tile)],
                        acc.at[pl.ds(r0, rows_per_tile)])

        base = wid * n_chunks
        pltpu.sync_copy(col_hbm.at[pl.ds(base, n_chunks)], col_v)
        pltpu.sync_copy(row_hbm.at[pl.ds(base, n_chunks)], row_v)
        pltpu.sync_copy(val_hbm.at[pl.ds(base, n_chunks)], val_v)
        plsc.subcore_barrier()

        def chunk_body(j, carry):
            pltpu.sync_copy(x_hbm.at[col_v.at[j]], rows_v)
            vrow = val_v.at[j]
            for g in range(CHUNK // LANES):
                v16 = vrow[pl.ds(g * LANES, LANES)]
                for l in range(LANES):
                    e = g * LANES + l
                    bc = _bcast(v16, l)
                    for k in range(d // LANES):
                        sl = pl.ds(k * LANES, LANES)
                        rows_v[e, sl] = rows_v[e, sl] * bc
            pltpu.sync_copy(rows_v, acc.at[row_v.at[j]], add=True)
            return carry

        lax.fori_loop(0, n_chunks, chunk_body, 0)
        plsc.subcore_barrier()

        pltpu.sync_copy(acc.at[pl.ds(r0, rows_per_tile)],
                        out_hbm.at[cid, pl.ds(r0, rows_per_tile)])

    return sc_aggregate


def _tc_combine_matmul(partials, w, n_nodes):
    d = partials.shape[2]

    def body(p_ref, w_ref, o_ref):
        agg = p_ref[0, :n_nodes] + p_ref[1, :n_nodes]
        o_ref[...] = lax.dot_general(
            agg, w_ref[...], (((1,), (1,)), ((), ())),
            preferred_element_type=jnp.float32)

    return pl.pallas_call(
        body,
        out_shape=jax.ShapeDtypeStruct((n_nodes, d), jnp.float32),
    )(partials, w)


def kernel(_x, adj_indices, adj_values, W):
    n_nodes, d = _x.shape
    e = adj_values.shape[0]

    row = adj_indices[0].astype(jnp.int32)
    col = adj_indices[1].astype(jnp.int32)
    val = adj_values.astype(jnp.float32)

    per_tile = -(-e // (NW * CHUNK * 8)) * CHUNK * 8
    e_pad = per_tile * NW
    n_chunks = per_tile // CHUNK
    pad = e_pad - e
    if pad:
        row = jnp.concatenate([row, jnp.zeros((pad,), jnp.int32)])
        col = jnp.concatenate([col, jnp.zeros((pad,), jnp.int32)])
        val = jnp.concatenate([val, jnp.zeros((pad,), jnp.float32)])
    row2 = row.reshape(NW * n_chunks, CHUNK)
    col2 = col.reshape(NW * n_chunks, CHUNK)
    val2 = val.reshape(NW * n_chunks, CHUNK)

    n_pad = -(-n_nodes // (8 * N_SUBCORES)) * 8 * N_SUBCORES
    zero = jnp.zeros((n_pad, d), jnp.float32)

    sc_aggregate = _make_sc_aggregate(n_pad, d, n_chunks)
    partials = sc_aggregate(_x, col2, row2, val2, zero)
    return _tc_combine_matmul(partials, W, n_nodes)


# speedup vs baseline: 3.1365x; 1.0675x over previous
import functools

import jax
import jax.numpy as jnp
from jax import lax
from jax.experimental import pallas as pl
from jax.experimental.pallas import tpu as pltpu
from jax.experimental.pallas import tpu_sc as plsc

N_CORES = 2
N_SUBCORES = 16
LANES = 16
CHUNK = 32
HALF = 5120
TRASH = HALF
ACC_R = HALF + 8

_GDN = lax.GatherDimensionNumbers(
    offset_dims=(), collapsed_slice_dims=(0,), start_index_map=(0,))


def _bcast(v16, lane):
    idx = jnp.full((LANES, 1), lane, dtype=jnp.int32)
    return lax.gather(v16, idx, _GDN, (1,),
                      mode=lax.GatherScatterMode.PROMISE_IN_BOUNDS)


def _make_sc_aggregate(n_xpad, d, n_chunks):
    x_rows_per_tile = n_xpad // N_SUBCORES
    out_rows_per_tile = HALF // N_SUBCORES
    mesh = plsc.VectorSubcoreMesh(
        core_axis_name="c", subcore_axis_name="s",
        num_cores=N_CORES, num_subcores=N_SUBCORES)

    @functools.partial(
        pl.kernel,
        out_type=jax.ShapeDtypeStruct((N_CORES, HALF, d), jnp.float32),
        mesh=mesh,
        scratch_types=[
            pltpu.VMEM((CHUNK, d), jnp.float32),
            pltpu.VMEM((CHUNK, d), jnp.float32),
            pltpu.VMEM((2, CHUNK), jnp.int32),
            pltpu.VMEM((2, CHUNK), jnp.float32),
            pltpu.VMEM((2, CHUNK), jnp.int32),
            pltpu.SemaphoreType.DMA,
            pltpu.SemaphoreType.DMA,
            pltpu.SemaphoreType.DMA,
            pltpu.SemaphoreType.DMA,
            pltpu.SemaphoreType.DMA,
            pltpu.SemaphoreType.DMA,
            pltpu.SemaphoreType.DMA,
            pltpu.SemaphoreType.DMA,
            pltpu.VMEM_SHARED((n_xpad, d), jnp.float32),
            pltpu.VMEM_SHARED((ACC_R, d), jnp.float32),
        ],
    )
    def sc_aggregate(x_hbm, col_hbm, val_hbm, dst_hbm, zero_hbm, out_hbm,
                     rows0, rows1, col_v, val_v, dst_v,
                     g0, g1, cs0, cs1, vs0, vs1, ds0, ds1,
                     x_sp, acc):
        rows = [rows0, rows1]
        gsem = [g0, g1]
        csem = [cs0, cs1]
        vsem = [vs0, vs1]
        dsem = [ds0, ds1]

        cid = lax.axis_index("c")
        sid = lax.axis_index("s")

        xr0 = sid * x_rows_per_tile
        pltpu.sync_copy(x_hbm.at[pl.ds(xr0, x_rows_per_tile)],
                        x_sp.at[pl.ds(xr0, x_rows_per_tile)])
        ar0 = sid * out_rows_per_tile
        pltpu.sync_copy(zero_hbm.at[pl.ds(ar0, out_rows_per_tile)],
                        acc.at[pl.ds(ar0, out_rows_per_tile)])
        plsc.subcore_barrier()

        base = sid * n_chunks

        def gc(c):
            return base + lax.rem(c, n_chunks)

        def start_fetch(c, q):
            pltpu.make_async_copy(col_hbm.at[gc(c)], col_v.at[q],
                                  csem[q]).start()
            pltpu.make_async_copy(val_hbm.at[gc(c)], val_v.at[q],
                                  vsem[q]).start()
            pltpu.make_async_copy(dst_hbm.at[cid, gc(c)], dst_v.at[q],
                                  dsem[q]).start()

        def wait_col(c, q):
            pltpu.make_async_copy(col_hbm.at[gc(c)], col_v.at[q],
                                  csem[q]).wait()

        def wait_valdst(c, q):
            pltpu.make_async_copy(val_hbm.at[gc(c)], val_v.at[q],
                                  vsem[q]).wait()
            pltpu.make_async_copy(dst_hbm.at[cid, gc(c)], dst_v.at[q],
                                  dsem[q]).wait()

        def start_g(q, b):
            pltpu.make_async_copy(x_sp.at[col_v.at[q]], rows[b],
                                  gsem[b]).start()

        def wait_g(q, b):
            pltpu.make_async_copy(x_sp.at[col_v.at[q]], rows[b],
                                  gsem[b]).wait()

        def scale(q, b):
            rb = rows[b]
            for g in range(CHUNK // LANES):
                v16 = val_v[q, pl.ds(g * LANES, LANES)]
                for l in range(LANES):
                    e = g * LANES + l
                    bc = _bcast(v16, l)
                    for k in range(d // LANES):
                        sl = pl.ds(k * LANES, LANES)
                        rb[e, sl] = rb[e, sl] * bc

        start_fetch(0, 0)
        start_fetch(1, 1)
        wait_col(0, 0)
        start_g(0, 0)

        def slot(c, b):
            q = b
            wait_col(c + 1, 1 - q)
            start_g(1 - q, 1 - b)
            wait_g(q, b)
            wait_valdst(c, q)
            scale(q, b)
            pltpu.sync_copy(rows[b], acc.at[dst_v.at[q]], add=True)
            start_fetch(c + 2, q)

        def pair(p, carry):
            c = 2 * p
            slot(c, 0)
            slot(c + 1, 1)
            return carry

        lax.fori_loop(0, n_chunks // 2, pair, 0)

        wait_g(0, 0)
        wait_col(n_chunks + 1, 1)
        wait_valdst(n_chunks, 0)
        wait_valdst(n_chunks + 1, 1)

        plsc.subcore_barrier()

        pltpu.sync_copy(acc.at[pl.ds(ar0, out_rows_per_tile)],
                        out_hbm.at[cid, pl.ds(ar0, out_rows_per_tile)])

    return sc_aggregate


def _tc_combine_matmul(partials, w, n_nodes):
    d = partials.shape[2]

    def body(p_ref, w_ref, o_ref):
        agg = jnp.concatenate(
            [p_ref[0], p_ref[1, :n_nodes - HALF]], axis=0)
        o_ref[...] = lax.dot_general(
            agg, w_ref[...], (((1,), (1,)), ((), ())),
            preferred_element_type=jnp.float32)

    return pl.pallas_call(
        body,
        out_shape=jax.ShapeDtypeStruct((n_nodes, d), jnp.float32),
    )(partials, w)


def kernel(_x, adj_indices, adj_values, W):
    n_nodes, d = _x.shape
    e = adj_values.shape[0]

    row = adj_indices[0].astype(jnp.int32)
    col = adj_indices[1].astype(jnp.int32)
    val = adj_values.astype(jnp.float32)

    per_tile = -(-e // (N_SUBCORES * CHUNK * 2)) * CHUNK * 2
    n_chunks = per_tile // CHUNK
    pad = per_tile * N_SUBCORES - e
    if pad:
        row = jnp.concatenate([row, jnp.zeros((pad,), jnp.int32)])
        col = jnp.concatenate([col, jnp.zeros((pad,), jnp.int32)])
        val = jnp.concatenate([val, jnp.zeros((pad,), jnp.float32)])

    col2 = col.reshape(-1, CHUNK)
    val2 = val.reshape(-1, CHUNK)
    dst0 = jnp.where(row < HALF, row, TRASH).reshape(-1, CHUNK)
    dst1 = jnp.where(row >= HALF, row - HALF, TRASH).reshape(-1, CHUNK)
    dsts = jnp.stack([dst0, dst1], axis=0)

    n_xpad = -(-n_nodes // (8 * N_SUBCORES)) * 8 * N_SUBCORES
    x_p = jnp.concatenate(
        [_x, jnp.zeros((n_xpad - n_nodes, d), jnp.float32)])
    zero = jnp.zeros((HALF, d), jnp.float32)

    sc_aggregate = _make_sc_aggregate(n_xpad, d, n_chunks)
    partials = sc_aggregate(x_p, col2, val2, dsts, zero)
    return _tc_combine_matmul(partials, W, n_nodes)


# speedup vs baseline: 4.1383x; 1.3194x over previous
import functools

import jax
import jax.numpy as jnp
from jax import lax
from jax.experimental import pallas as pl
from jax.experimental.pallas import tpu as pltpu
from jax.experimental.pallas import tpu_sc as plsc

N_CORES = 2
N_SUBCORES = 16
LANES = 16
CHUNK = 128
HALF = 5120
TRASH = HALF
ACC_R = HALF + 8

_GDN = lax.GatherDimensionNumbers(
    offset_dims=(), collapsed_slice_dims=(0,), start_index_map=(0,))


def _bcast(v16, lane):
    idx = jnp.full((LANES, 1), lane, dtype=jnp.int32)
    return lax.gather(v16, idx, _GDN, (1,),
                      mode=lax.GatherScatterMode.PROMISE_IN_BOUNDS)


def _make_sc_aggregate(n_xpad, d, n_chunks):
    dp = d // 2
    x_rows_per_tile = n_xpad // N_SUBCORES
    out_rows_per_tile = HALF // N_SUBCORES
    mesh = plsc.VectorSubcoreMesh(
        core_axis_name="c", subcore_axis_name="s",
        num_cores=N_CORES, num_subcores=N_SUBCORES)

    @functools.partial(
        pl.kernel,
        out_type=jax.ShapeDtypeStruct((N_CORES, HALF, d), jnp.float32),
        mesh=mesh,
        scratch_types=[
            pltpu.VMEM((CHUNK, dp), jnp.int32),
            pltpu.VMEM((CHUNK, d), jnp.float32),
            pltpu.VMEM((2, 3, CHUNK), jnp.int32),
            pltpu.VMEM((2, CHUNK), jnp.float32),
            pltpu.SemaphoreType.DMA,
            pltpu.SemaphoreType.DMA,
            pltpu.SemaphoreType.DMA,
            pltpu.SemaphoreType.DMA,
            pltpu.VMEM_SHARED((n_xpad, dp), jnp.int32),
            pltpu.VMEM_SHARED((ACC_R, d), jnp.float32),
        ],
    )
    def sc_aggregate(x_hbm, meta_hbm, val_hbm, zero_hbm, out_hbm,
                     rows0, scaled, meta_v, val_v,
                     ms0, ms1, vs0, vs1,
                     x_sp, acc):
        msem = [ms0, ms1]
        vsem = [vs0, vs1]

        cid = lax.axis_index("c")
        sid = lax.axis_index("s")

        xr0 = sid * x_rows_per_tile
        pltpu.sync_copy(x_hbm.at[pl.ds(xr0, x_rows_per_tile)],
                        x_sp.at[pl.ds(xr0, x_rows_per_tile)])
        ar0 = sid * out_rows_per_tile
        pltpu.sync_copy(zero_hbm.at[pl.ds(ar0, out_rows_per_tile)],
                        acc.at[pl.ds(ar0, out_rows_per_tile)])
        plsc.subcore_barrier()

        base = sid * n_chunks

        def gc(c):
            return base + lax.rem(c, n_chunks)

        def start_fetch(c, q):
            pltpu.make_async_copy(meta_hbm.at[gc(c)], meta_v.at[q],
                                  msem[q]).start()
            pltpu.make_async_copy(val_hbm.at[gc(c)], val_v.at[q],
                                  vsem[q]).start()

        def wait_meta(c, q):
            pltpu.make_async_copy(meta_hbm.at[gc(c)], meta_v.at[q],
                                  msem[q]).wait()

        def wait_val(c, q):
            pltpu.make_async_copy(val_hbm.at[gc(c)], val_v.at[q],
                                  vsem[q]).wait()

        def sync_g(q):
            pltpu.sync_copy(x_sp.at[meta_v.at[q, 0]], rows0)

        himask = jnp.full((LANES,), -65536, dtype=jnp.int32)

        def scale(q):
            rb = rows0
            for g in range(CHUNK // LANES):
                v16 = val_v[q, pl.ds(g * LANES, LANES)]
                for l in range(LANES):
                    e = g * LANES + l
                    bc = _bcast(v16, l)
                    for k in range(dp // LANES):
                        sl = pl.ds(k * LANES, LANES)
                        vi = rb[e, sl]
                        lo = lax.bitcast_convert_type(vi << 16, jnp.float32)
                        hi = lax.bitcast_convert_type(vi & himask, jnp.float32)
                        scaled[e, sl] = lo * bc
                        scaled[e, pl.ds(dp + k * LANES, LANES)] = hi * bc

        start_fetch(0, 0)
        start_fetch(1, 1)

        def slot(c, q):
            wait_meta(c, q)
            sync_g(q)
            wait_val(c, q)
            scale(q)
            pltpu.sync_copy(scaled, acc.at[meta_v.at[q, 1 + cid]], add=True)
            start_fetch(c + 2, q)

        def pair(p, carry):
            c = 2 * p
            slot(c, 0)
            slot(c + 1, 1)
            return carry

        lax.fori_loop(0, n_chunks // 2, pair, 0)

        wait_meta(n_chunks, 0)
        wait_val(n_chunks, 0)
        wait_meta(n_chunks + 1, 1)
        wait_val(n_chunks + 1, 1)

        plsc.subcore_barrier()

        pltpu.sync_copy(acc.at[pl.ds(ar0, out_rows_per_tile)],
                        out_hbm.at[cid, pl.ds(ar0, out_rows_per_tile)])

    return sc_aggregate


def _tc_combine_matmul(partials, w, n_nodes):
    d = partials.shape[2]

    def body(p_ref, w_ref, o_ref):
        agg = jnp.concatenate(
            [p_ref[0], p_ref[1, :n_nodes - HALF]], axis=0)
        o_ref[...] = lax.dot_general(
            agg, w_ref[...], (((1,), (1,)), ((), ())),
            preferred_element_type=jnp.float32)

    return pl.pallas_call(
        body,
        out_shape=jax.ShapeDtypeStruct((n_nodes, d), jnp.float32),
    )(partials, w)


def kernel(_x, adj_indices, adj_values, W):
    n_nodes, d = _x.shape
    e = adj_values.shape[0]

    row = adj_indices[0].astype(jnp.int32)
    col = adj_indices[1].astype(jnp.int32)
    val = adj_values.astype(jnp.float32)

    per_tile = -(-e // (N_SUBCORES * CHUNK * 2)) * CHUNK * 2
    n_chunks = per_tile // CHUNK
    pad = per_tile * N_SUBCORES - e
    if pad:
        row = jnp.concatenate([row, jnp.zeros((pad,), jnp.int32)])
        col = jnp.concatenate([col, jnp.zeros((pad,), jnp.int32)])
        val = jnp.concatenate([val, jnp.zeros((pad,), jnp.float32)])

    dst0 = jnp.where(row < HALF, row, TRASH)
    dst1 = jnp.where(row >= HALF, row - HALF, TRASH)
    meta = jnp.stack(
        [col.reshape(-1, CHUNK), dst0.reshape(-1, CHUNK),
         dst1.reshape(-1, CHUNK)], axis=1)
    val2 = val.reshape(-1, CHUNK)

    n_xpad = -(-n_nodes // (8 * N_SUBCORES)) * 8 * N_SUBCORES
    x_p = jnp.concatenate(
        [_x, jnp.zeros((n_xpad - n_nodes, d), jnp.float32)])
    x_bf = x_p.astype(jnp.bfloat16)
    lo = lax.bitcast_convert_type(x_bf[:, :d // 2], jnp.uint16)
    hi = lax.bitcast_convert_type(x_bf[:, d // 2:], jnp.uint16)
    x_packed = lo.astype(jnp.int32) | (hi.astype(jnp.int32) << 16)

    zero = jnp.zeros((HALF, d), jnp.float32)

    sc_aggregate = _make_sc_aggregate(n_xpad, d, n_chunks)
    partials = sc_aggregate(x_packed, meta, val2, zero)
    return _tc_combine_matmul(partials, W, n_nodes)


# speedup vs baseline: 4.3881x; 1.0604x over previous
import functools

import jax
import jax.numpy as jnp
from jax import lax
from jax.experimental import pallas as pl
from jax.experimental.pallas import tpu as pltpu
from jax.experimental.pallas import tpu_sc as plsc

N_CORES = 2
N_SUBCORES = 16
LANES = 16
CHUNK = 64
HALF = 5120
TRASH = HALF
ACC_R = HALF + 8

_GDN = lax.GatherDimensionNumbers(
    offset_dims=(), collapsed_slice_dims=(0,), start_index_map=(0,))


def _bcast(v16, lane):
    idx = jnp.full((LANES, 1), lane, dtype=jnp.int32)
    return lax.gather(v16, idx, _GDN, (1,),
                      mode=lax.GatherScatterMode.PROMISE_IN_BOUNDS)


def _make_sc_aggregate(n_xpad, d, n_chunks):
    x_rows_per_tile = n_xpad // N_SUBCORES
    out_rows_per_tile = HALF // N_SUBCORES
    mesh = plsc.VectorSubcoreMesh(
        core_axis_name="c", subcore_axis_name="s",
        num_cores=N_CORES, num_subcores=N_SUBCORES)

    @functools.partial(
        pl.kernel,
        out_type=jax.ShapeDtypeStruct((N_CORES, HALF, d), jnp.float32),
        mesh=mesh,
        scratch_types=[
            pltpu.VMEM((CHUNK, d), jnp.float32),
            pltpu.VMEM((2, CHUNK), jnp.int32),
            pltpu.VMEM((2, CHUNK), jnp.float32),
            pltpu.VMEM((2, CHUNK), jnp.int32),
            pltpu.SemaphoreType.DMA,
            pltpu.SemaphoreType.DMA,
            pltpu.SemaphoreType.DMA,
            pltpu.SemaphoreType.DMA,
            pltpu.SemaphoreType.DMA,
            pltpu.SemaphoreType.DMA,
            pltpu.VMEM_SHARED((n_xpad, d), jnp.float32),
            pltpu.VMEM_SHARED((ACC_R, d), jnp.float32),
        ],
    )
    def sc_aggregate(x_hbm, col_hbm, val_hbm, dst_hbm, zero_hbm, out_hbm,
                     rows0, col_v, val_v, dst_v,
                     cs0, cs1, vs0, vs1, ds0, ds1,
                     x_sp, acc):
        csem = [cs0, cs1]
        vsem = [vs0, vs1]
        dsem = [ds0, ds1]

        cid = lax.axis_index("c")
        sid = lax.axis_index("s")

        xr0 = sid * x_rows_per_tile
        pltpu.sync_copy(x_hbm.at[pl.ds(xr0, x_rows_per_tile)],
                        x_sp.at[pl.ds(xr0, x_rows_per_tile)])
        ar0 = sid * out_rows_per_tile
        pltpu.sync_copy(zero_hbm.at[pl.ds(ar0, out_rows_per_tile)],
                        acc.at[pl.ds(ar0, out_rows_per_tile)])
        plsc.subcore_barrier()

        base = sid * n_chunks

        def gc(c):
            return base + lax.rem(c, n_chunks)

        def start_fetch(c, q):
            pltpu.make_async_copy(col_hbm.at[gc(c)], col_v.at[q],
                                  csem[q]).start()
            pltpu.make_async_copy(val_hbm.at[gc(c)], val_v.at[q],
                                  vsem[q]).start()
            pltpu.make_async_copy(dst_hbm.at[cid, gc(c)], dst_v.at[q],
                                  dsem[q]).start()

        def wait_col(c, q):
            pltpu.make_async_copy(col_hbm.at[gc(c)], col_v.at[q],
                                  csem[q]).wait()

        def wait_valdst(c, q):
            pltpu.make_async_copy(val_hbm.at[gc(c)], val_v.at[q],
                                  vsem[q]).wait()
            pltpu.make_async_copy(dst_hbm.at[cid, gc(c)], dst_v.at[q],
                                  dsem[q]).wait()

        def sync_g(q):
            pltpu.sync_copy(x_sp.at[col_v.at[q]], rows0)

        def scale(q):
            rb = rows0
            for g in range(CHUNK // LANES):
                v16 = val_v[q, pl.ds(g * LANES, LANES)]
                for l in range(LANES):
                    e = g * LANES + l
                    bc = _bcast(v16, l)
                    for k in range(d // LANES):
                        sl = pl.ds(k * LANES, LANES)
                        rb[e, sl] = rb[e, sl] * bc

        start_fetch(0, 0)
        start_fetch(1, 1)

        def slot(c, q):
            wait_col(c, q)
            sync_g(q)
            wait_valdst(c, q)
            scale(q)
            pltpu.sync_copy(rows0, acc.at[dst_v.at[q]], add=True)
            start_fetch(c + 2, q)

        def pair(p, carry):
            c = 2 * p
            slot(c, 0)
            slot(c + 1, 1)
            return carry

        lax.fori_loop(0, n_chunks // 2, pair, 0)

        wait_col(n_chunks, 0)
        wait_valdst(n_chunks, 0)
        wait_col(n_chunks + 1, 1)
        wait_valdst(n_chunks + 1, 1)

        plsc.subcore_barrier()

        pltpu.sync_copy(acc.at[pl.ds(ar0, out_rows_per_tile)],
                        out_hbm.at[cid, pl.ds(ar0, out_rows_per_tile)])

    return sc_aggregate


def _tc_combine_matmul(partials, w, n_nodes):
    d = partials.shape[2]

    def body(p_ref, w_ref, o_ref):
        agg = jnp.concatenate(
            [p_ref[0], p_ref[1, :n_nodes - HALF]], axis=0)
        o_ref[...] = lax.dot_general(
            agg, w_ref[...], (((1,), (1,)), ((), ())),
            preferred_element_type=jnp.float32)

    return pl.pallas_call(
        body,
        out_shape=jax.ShapeDtypeStruct((n_nodes, d), jnp.float32),
    )(partials, w)


def kernel(_x, adj_indices, adj_values, W):
    n_nodes, d = _x.shape
    e = adj_values.shape[0]

    row = adj_indices[0].astype(jnp.int32)
    col = adj_indices[1].astype(jnp.int32)
    val = adj_values.astype(jnp.float32)

    per_tile = -(-e // (N_SUBCORES * CHUNK * 2)) * CHUNK * 2
    n_chunks = per_tile // CHUNK
    pad = per_tile * N_SUBCORES - e
    if pad:
        row = jnp.concatenate([row, jnp.zeros((pad,), jnp.int32)])
        col = jnp.concatenate([col, jnp.zeros((pad,), jnp.int32)])
        val = jnp.concatenate([val, jnp.zeros((pad,), jnp.float32)])

    col2 = col.reshape(-1, CHUNK)
    val2 = val.reshape(-1, CHUNK)
    dst0 = jnp.where(row < HALF, row, TRASH).reshape(-1, CHUNK)
    dst1 = jnp.where(row >= HALF, row - HALF, TRASH).reshape(-1, CHUNK)
    dsts = jnp.stack([dst0, dst1], axis=0)

    n_xpad = -(-n_nodes // (8 * N_SUBCORES)) * 8 * N_SUBCORES
    x_p = jnp.concatenate(
        [_x, jnp.zeros((n_xpad - n_nodes, d), jnp.float32)])
    zero = jnp.zeros((HALF, d), jnp.float32)

    sc_aggregate = _make_sc_aggregate(n_xpad, d, n_chunks)
    partials = sc_aggregate(x_p, col2, val2, dsts, zero)
    return _tc_combine_matmul(partials, W, n_nodes)
